# baseline (speedup 1.0000x reference)
import functools

import jax
import jax.numpy as jnp
from jax import lax
from jax.experimental import pallas as pl
from jax.experimental.pallas import tpu as pltpu
from jax.experimental.pallas import tpu_sc as plsc

B = 16384
D = 32
L = 16
NC = 2
NS = 16
NW = NC * NS
BPW = B // NW
CHUNK = 128
NCHUNK = BPW // CHUNK
GPC = CHUNK // L
PAD_IDX = 0


def _emb_body(idx_hbm, table_hbm, out_hbm, idx2d, rows_v, sem):
    wid = lax.axis_index("s") * NC + lax.axis_index("c")
    base = wid * BPW

    for c in range(NCHUNK):
        pltpu.sync_copy(idx_hbm.at[pl.ds(base + c * CHUNK, CHUNK)], idx2d.at[c])

    copies = []
    for c in range(NCHUNK):
        copies.append(
            pltpu.async_copy(
                table_hbm.at[idx2d.at[c]],
                rows_v.at[pl.ds(c * CHUNK, CHUNK)],
                sem,
            )
        )
    for cp in copies:
        cp.wait()

    zero = jnp.zeros((L,), jnp.float32)
    for c in range(NCHUNK):
        idx_row = idx2d.at[c]

        def fix_group(g, carry, idx_row=idx_row, c=c):
            idxv = idx_row[pl.ds(g * L, L)]
            m = jnp.min(idxv)

            @pl.when(m == jnp.int32(PAD_IDX))
            def _():
                zm = idxv == jnp.int32(PAD_IDX)
                rows = lax.iota(jnp.int32, L) + (c * CHUNK + g * L)
                for j in range(D):
                    col = jnp.full((L,), j, jnp.int32)
                    plsc.store_scatter(rows_v, [rows, col], zero, mask=zm)

            return carry

        lax.fori_loop(0, GPC, fix_group, 0)

    pltpu.sync_copy(rows_v, out_hbm.at[pl.ds(base, BPW)])


@functools.partial(
    pl.kernel,
    mesh=plsc.VectorSubcoreMesh(core_axis_name="c", subcore_axis_name="s"),
    out_type=jax.ShapeDtypeStruct((B, D), jnp.float32),
    scratch_types=[
        pltpu.VMEM((NCHUNK, CHUNK), jnp.int32),
        pltpu.VMEM((BPW, D), jnp.float32),
        pltpu.SemaphoreType.DMA,
    ],
    compiler_params=pltpu.CompilerParams(
        use_tc_tiling_on_sc=False,
        needs_layout_passes=False,
    ),
)
def _emb(idx_hbm, table_hbm, out_hbm, idx2d, rows_v, sem):
    _emb_body(idx_hbm, table_hbm, out_hbm, idx2d, rows_v, sem)


def kernel(user_id, id_table):
    return _emb(user_id.astype(jnp.int32), id_table)
